# Initial kernel scaffold; baseline (speedup 1.0000x reference)
import jax
import jax.numpy as jnp
from jax.experimental import pallas as pl


def kernel(x, edge_index, edge_attr, batch, W1, b1, W2, b2, W3, b3, Wlin, blin):
    raise NotImplementedError("write your pallas kernel here")


# speedup vs baseline: 3.4448x; 3.4448x over previous
import functools

import jax
import jax.numpy as jnp
from jax import lax
from jax.experimental import pallas as pl
from jax.experimental.pallas import tpu as pltpu
from jax.experimental.pallas import tpu_sc as plsc

_N = 10000
_E = 320000
_DIN = 128
_DE = 16
_H = 256
_C = 10
_G = 64

_NC = 2
_NS = 16
_NP = 10240
_ROWS_PT = _NP // _NS
_EPT = 20480
_CH = 128
_NCH = _EPT // _CH
_W1 = 144
_W23 = 128
_BLK = 1024


def _sc_mesh():
    return plsc.VectorSubcoreMesh(
        core_axis_name="c", subcore_axis_name="s", num_cores=_NC, num_subcores=_NS
    )


_SC_PARAMS = pltpu.CompilerParams(use_tc_tiling_on_sc=False)


def _sc_spmm_first(table, src_ids, dst_ids, ea_ids, zeros_w, zeros_e):
    @functools.partial(
        pl.kernel,
        out_type=(
            jax.ShapeDtypeStruct((_NC * _NP, _W1), jnp.float32),
            jax.ShapeDtypeStruct((_NP, _DE), jnp.float32),
        ),
        mesh=_sc_mesh(),
        compiler_params=_SC_PARAMS,
        scratch_types=(
            pltpu.VMEM_SHARED((_NP, _W1), jnp.float32),
            pltpu.VMEM_SHARED((_NP, _DE), jnp.float32),
            pltpu.VMEM((_CH,), jnp.int32),
            pltpu.VMEM((_CH,), jnp.int32),
            pltpu.VMEM((_CH, _W1), jnp.float32),
            pltpu.VMEM((_CH, _DE), jnp.float32),
            pltpu.SemaphoreType.DMA,
        ),
    )
    def body(table_h, src_h, dst_h, ea_h, z_h, ze_h, out_h, eagg_h,
             acc, acc_e, srcv, dstv, gbuf, eabuf, sem):
        cid = lax.axis_index("c")
        sid = lax.axis_index("s")
        wid = cid * _NS + sid
        r0 = sid * _ROWS_PT
        pltpu.sync_copy(z_h.at[pl.ds(r0, _ROWS_PT)], acc.at[pl.ds(r0, _ROWS_PT)])

        @pl.when(cid == 0)
        def _():
            pltpu.sync_copy(ze_h.at[pl.ds(r0, _ROWS_PT)],
                            acc_e.at[pl.ds(r0, _ROWS_PT)])

        plsc.subcore_barrier()

        def step(j, carry):
            pltpu.sync_copy(src_h.at[wid * _NCH + j], srcv)
            pltpu.sync_copy(dst_h.at[sid * _NCH + j], dstv)
            pltpu.async_copy(table_h.at[srcv], gbuf, sem).wait()
            pltpu.sync_copy(gbuf, acc.at[dstv], add=True)

            @pl.when(cid == 0)
            def _():
                pltpu.sync_copy(ea_h.at[sid * _NCH + j], eabuf)
                pltpu.sync_copy(eabuf, acc_e.at[dstv], add=True)

            return carry

        lax.fori_loop(0, _NCH, step, 0)
        plsc.subcore_barrier()
        pltpu.sync_copy(acc.at[pl.ds(r0, _ROWS_PT)],
                        out_h.at[pl.ds(cid * _NP + r0, _ROWS_PT)])

        @pl.when(cid == 0)
        def _():
            pltpu.sync_copy(acc_e.at[pl.ds(r0, _ROWS_PT)],
                            eagg_h.at[pl.ds(r0, _ROWS_PT)])

    return body(table, src_ids, dst_ids, ea_ids, zeros_w, zeros_e)


def _sc_spmm(table, src_ids, dst_ids, zeros_w):
    @functools.partial(
        pl.kernel,
        out_type=jax.ShapeDtypeStruct((_NC * _NP, _W23), jnp.float32),
        mesh=_sc_mesh(),
        compiler_params=_SC_PARAMS,
        scratch_types=(
            pltpu.VMEM_SHARED((_NP, _W23), jnp.float32),
            pltpu.VMEM((_CH,), jnp.int32),
            pltpu.VMEM((_CH,), jnp.int32),
            pltpu.VMEM((_CH, _W23), jnp.float32),
            pltpu.SemaphoreType.DMA,
        ),
    )
    def body(table_h, src_h, dst_h, z_h, out_h, acc, srcv, dstv, gbuf, sem):
        cid = lax.axis_index("c")
        sid = lax.axis_index("s")
        wid = cid * _NS + sid
        r0 = sid * _ROWS_PT
        pltpu.sync_copy(z_h.at[pl.ds(r0, _ROWS_PT)], acc.at[pl.ds(r0, _ROWS_PT)])
        plsc.subcore_barrier()

        def step(j, carry):
            pltpu.sync_copy(src_h.at[wid * _NCH + j], srcv)
            pltpu.sync_copy(dst_h.at[sid * _NCH + j], dstv)
            pltpu.async_copy(table_h.at[srcv], gbuf, sem).wait()
            pltpu.sync_copy(gbuf, acc.at[dstv], add=True)
            return carry

        lax.fori_loop(0, _NCH, step, 0)
        plsc.subcore_barrier()
        pltpu.sync_copy(acc.at[pl.ds(r0, _ROWS_PT)],
                        out_h.at[pl.ds(cid * _NP + r0, _ROWS_PT)])

    return body(table, src_ids, dst_ids, zeros_w)


def _tc_mm(x, w):
    m, k = x.shape
    n = w.shape[1]

    def kern(x_ref, w_ref, o_ref):
        o_ref[...] = jnp.dot(x_ref[...], w_ref[...],
                             preferred_element_type=jnp.float32)

    return pl.pallas_call(
        kern,
        grid=(m // _BLK,),
        in_specs=[
            pl.BlockSpec((_BLK, k), lambda i: (i, 0)),
            pl.BlockSpec((k, n), lambda i: (0, 0)),
        ],
        out_specs=pl.BlockSpec((_BLK, n), lambda i: (i, 0)),
        out_shape=jax.ShapeDtypeStruct((m, n), jnp.float32),
    )(x, w)


def _tc_combine(p, s, deg, eagg, we_t, b, wn):
    m, h = p.shape
    n = wn.shape[1]

    def kern(p_ref, s_ref, d_ref, e_ref, we_ref, b_ref, wn_ref, o_ref):
        r = jnp.dot(e_ref[...], we_ref[...], preferred_element_type=jnp.float32)
        hcur = jnp.maximum(d_ref[...] * (p_ref[...] + b_ref[...])
                           + s_ref[...] + r, 0.0)
        o_ref[...] = jnp.dot(hcur, wn_ref[...],
                             preferred_element_type=jnp.float32)

    return pl.pallas_call(
        kern,
        grid=(m // _BLK,),
        in_specs=[
            pl.BlockSpec((_BLK, h), lambda i: (i, 0)),
            pl.BlockSpec((_BLK, h), lambda i: (i, 0)),
            pl.BlockSpec((_BLK, 1), lambda i: (i, 0)),
            pl.BlockSpec((_BLK, _DE), lambda i: (i, 0)),
            pl.BlockSpec((_DE, h), lambda i: (0, 0)),
            pl.BlockSpec((1, h), lambda i: (0, 0)),
            pl.BlockSpec((h, n), lambda i: (0, 0)),
        ],
        out_specs=pl.BlockSpec((_BLK, n), lambda i: (i, 0)),
        out_shape=jax.ShapeDtypeStruct((m, n), jnp.float32),
    )(p, s, deg, eagg, we_t, b, wn)


def _tc_final(p, s, deg, eagg, we_t, b, batch, wl_t, bl):
    m, h = p.shape
    nblk = m // _BLK

    def kern(p_ref, s_ref, d_ref, e_ref, we_ref, b_ref, batch_ref, wl_ref,
             bl_ref, o_ref, sums, cnts):
        i = pl.program_id(0)

        @pl.when(i == 0)
        def _():
            sums[...] = jnp.zeros_like(sums)
            cnts[...] = jnp.zeros_like(cnts)

        r = jnp.dot(e_ref[...], we_ref[...], preferred_element_type=jnp.float32)
        hcur = jnp.maximum(d_ref[...] * (p_ref[...] + b_ref[...])
                           + s_ref[...] + r, 0.0)
        oh = (batch_ref[...][None, :]
              == lax.broadcasted_iota(jnp.int32, (_G, _BLK), 0)
              ).astype(jnp.float32)
        sums[...] += jnp.dot(oh, hcur, preferred_element_type=jnp.float32)
        cnts[...] += jnp.sum(oh, axis=1, keepdims=True)

        @pl.when(i == nblk - 1)
        def _():
            pooled = sums[...] / jnp.maximum(cnts[...], 1.0)
            logits = jnp.dot(pooled, wl_ref[...],
                             preferred_element_type=jnp.float32) + bl_ref[...]
            mx = jnp.max(logits, axis=1, keepdims=True)
            lse = jnp.log(jnp.sum(jnp.exp(logits - mx), axis=1, keepdims=True))
            o_ref[...] = (logits - mx) - lse

    return pl.pallas_call(
        kern,
        grid=(nblk,),
        in_specs=[
            pl.BlockSpec((_BLK, h), lambda i: (i, 0)),
            pl.BlockSpec((_BLK, h), lambda i: (i, 0)),
            pl.BlockSpec((_BLK, 1), lambda i: (i, 0)),
            pl.BlockSpec((_BLK, _DE), lambda i: (i, 0)),
            pl.BlockSpec((_DE, h), lambda i: (0, 0)),
            pl.BlockSpec((1, h), lambda i: (0, 0)),
            pl.BlockSpec((_BLK,), lambda i: (i,)),
            pl.BlockSpec((h, _C), lambda i: (0, 0)),
            pl.BlockSpec((1, _C), lambda i: (0, 0)),
        ],
        out_specs=pl.BlockSpec((_G, _C), lambda i: (0, 0)),
        out_shape=jax.ShapeDtypeStruct((_G, _C), jnp.float32),
        scratch_shapes=[
            pltpu.VMEM((_G, h), jnp.float32),
            pltpu.VMEM((_G, 1), jnp.float32),
        ],
    )(p, s, deg, eagg, we_t, b, batch, wl_t, bl)


def _mk_table1(q, ones_col, zpad):
    qa = jnp.concatenate([q[:, :128], ones_col, zpad], axis=1)
    qb = jnp.concatenate([q[:, 128:], ones_col, zpad], axis=1)
    return jnp.concatenate([qa, qb], axis=0)


def _mk_table23(q):
    return jnp.concatenate([q[:, :128], q[:, 128:]], axis=0)


def kernel(x, edge_index, edge_attr, batch, W1, b1, W2, b2, W3, b3, Wlin, blin):
    src = edge_index[0]
    dst = edge_index[1]

    xp = jnp.pad(x, ((0, _NP - _N), (0, 0)))
    batch_p = jnp.pad(batch, (0, _NP - _N), constant_values=_G)

    ept_real = _E // _NS
    src_t = jnp.pad(src.reshape(_NS, ept_real), ((0, 0), (0, _EPT - ept_real)))
    dst_t = jnp.pad(dst.reshape(_NS, ept_real), ((0, 0), (0, _EPT - ept_real)),
                    constant_values=_NP - 1)
    ea_t = jnp.pad(edge_attr.reshape(_NS, ept_real, _DE),
                   ((0, 0), (0, _EPT - ept_real), (0, 0)))
    src_ids = jnp.concatenate([src_t, src_t + _NP], axis=0)
    src_ids = src_ids.reshape(_NC * _NS * _NCH, _CH)
    dst_ids = dst_t.reshape(_NS * _NCH, _CH)
    ea_ids = ea_t.reshape(_NS * _NCH, _CH, _DE)

    zeros_w1 = jnp.zeros((_NP, _W1), jnp.float32)
    zeros_w23 = jnp.zeros((_NP, _W23), jnp.float32)
    zeros_e = jnp.zeros((_NP, _DE), jnp.float32)
    ones_col = jnp.ones((_NP, 1), jnp.float32)
    zpad = jnp.zeros((_NP, _W1 - 129), jnp.float32)

    w1i, w1j, w1e = W1[:, :_DIN], W1[:, _DIN:2 * _DIN], W1[:, 2 * _DIN:]
    w2i, w2j, w2e = W2[:, :_H], W2[:, _H:2 * _H], W2[:, 2 * _H:]
    w3i, w3j, w3e = W3[:, :_H], W3[:, _H:2 * _H], W3[:, 2 * _H:]

    m1 = _tc_mm(xp, jnp.concatenate([w1i.T, w1j.T], axis=1))
    t1 = _mk_table1(m1[:, _H:], ones_col, zpad)
    s1cat, eagg = _sc_spmm_first(t1, src_ids, dst_ids, ea_ids, zeros_w1, zeros_e)
    s1 = jnp.concatenate([s1cat[:_NP, :128], s1cat[_NP:, :128]], axis=1)
    deg = s1cat[:_NP, 128:129]

    m2 = _tc_combine(m1[:, :_H], s1, deg, eagg, w1e.T, b1.reshape(1, _H),
                     jnp.concatenate([w2i.T, w2j.T], axis=1))
    t2 = _mk_table23(m2[:, _H:])
    s2cat = _sc_spmm(t2, src_ids, dst_ids, zeros_w23)
    s2 = jnp.concatenate([s2cat[:_NP, :], s2cat[_NP:, :]], axis=1)

    m3 = _tc_combine(m2[:, :_H], s2, deg, eagg, w2e.T, b2.reshape(1, _H),
                     jnp.concatenate([w3i.T, w3j.T], axis=1))
    t3 = _mk_table23(m3[:, _H:])
    s3cat = _sc_spmm(t3, src_ids, dst_ids, zeros_w23)
    s3 = jnp.concatenate([s3cat[:_NP, :], s3cat[_NP:, :]], axis=1)

    return _tc_final(m3[:, :_H], s3, deg, eagg, w3e.T, b3.reshape(1, _H),
                     batch_p, Wlin.T, blin.reshape(1, _C))


# speedup vs baseline: 5.2657x; 1.5286x over previous
import functools

import jax
import jax.numpy as jnp
from jax import lax
from jax.experimental import pallas as pl
from jax.experimental.pallas import tpu as pltpu
from jax.experimental.pallas import tpu_sc as plsc

_N = 10000
_E = 320000
_DIN = 128
_DE = 16
_H = 256
_C = 10
_G = 64

_NC = 2
_NS = 16
_NP = 10240
_ROWS_PT = _NP // _NS
_EPT = 20480
_CH = 128
_NCH = _EPT // _CH
_STG = 40
_WS = 128
_WE = 32
_BLK = 1024


def _sc_mesh():
    return plsc.VectorSubcoreMesh(
        core_axis_name="c", subcore_axis_name="s", num_cores=_NC, num_subcores=_NS
    )


_SC_PARAMS = pltpu.CompilerParams(use_tc_tiling_on_sc=False)


def _sc_ea(ea_ids, dst_ids, zeros_e):
    @functools.partial(
        pl.kernel,
        out_type=jax.ShapeDtypeStruct((_NC * _NP, _WE), jnp.float32),
        mesh=_sc_mesh(),
        compiler_params=_SC_PARAMS,
        scratch_types=(
            pltpu.VMEM_SHARED((_NP, _WE), jnp.float32),
            pltpu.VMEM((_CH,), jnp.int32),
            pltpu.VMEM((_CH, _WE), jnp.float32),
        ),
    )
    def body(ea_h, dst_h, z_h, out_h, acc, didx, pbuf):
        cid = lax.axis_index("c")
        sid = lax.axis_index("s")
        r0 = sid * _ROWS_PT
        pltpu.sync_copy(z_h.at[pl.ds(r0, _ROWS_PT)], acc.at[pl.ds(r0, _ROWS_PT)])
        plsc.subcore_barrier()
        half = _NCH // 2

        def step(j, c):
            base = sid * _NCH + cid * half + j
            pltpu.sync_copy(dst_h.at[base], didx)
            pltpu.sync_copy(ea_h.at[base], pbuf)
            pltpu.sync_copy(pbuf, acc.at[didx], add=True)
            return c

        lax.fori_loop(0, half, step, 0)
        plsc.subcore_barrier()
        pltpu.sync_copy(acc.at[pl.ds(r0, _ROWS_PT)],
                        out_h.at[pl.ds(cid * _NP + r0, _ROWS_PT)])

    return body(ea_ids, dst_ids, zeros_e)


def _sc_spmm(table, src_ids, dst_ids, zeros_w):
    @functools.partial(
        pl.kernel,
        out_type=jax.ShapeDtypeStruct((_NC * _NP, _WS), jnp.float32),
        mesh=_sc_mesh(),
        compiler_params=_SC_PARAMS,
        scratch_types=(
            pltpu.VMEM_SHARED((_NP, _WS), jnp.float32),
            pltpu.VMEM((_STG, _CH), jnp.int32),
            pltpu.VMEM((_STG, _CH), jnp.int32),
            pltpu.VMEM((_CH, _WS), jnp.float32),
            pltpu.VMEM((_CH, _WS), jnp.float32),
            pltpu.SemaphoreType.DMA,
            pltpu.SemaphoreType.DMA,
        ),
    )
    def body(table_h, src_h, dst_h, z_h, out_h,
             acc, sidx, didx, gb0, gb1, sem0, sem1):
        cid = lax.axis_index("c")
        sid = lax.axis_index("s")
        wid = cid * _NS + sid
        r0 = sid * _ROWS_PT
        pltpu.sync_copy(z_h.at[pl.ds(r0, _ROWS_PT)], acc.at[pl.ds(r0, _ROWS_PT)])
        plsc.subcore_barrier()

        for t in range(_NCH // _STG):
            pltpu.sync_copy(src_h.at[pl.ds(wid * _NCH + t * _STG, _STG)], sidx)
            pltpu.sync_copy(dst_h.at[pl.ds(sid * _NCH + t * _STG, _STG)], didx)
            pltpu.async_copy(table_h.at[sidx.at[0]], gb0, sem0)

            def step(k, c):
                pltpu.async_copy(table_h.at[sidx.at[2 * k + 1]], gb1, sem1)
                pltpu.make_async_copy(table_h.at[sidx.at[2 * k]], gb0,
                                      sem0).wait()
                pltpu.sync_copy(gb0, acc.at[didx.at[2 * k]], add=True)

                @pl.when(k < _STG // 2 - 1)
                def _():
                    pltpu.async_copy(table_h.at[sidx.at[2 * k + 2]], gb0, sem0)

                pltpu.make_async_copy(table_h.at[sidx.at[2 * k + 1]], gb1,
                                      sem1).wait()
                pltpu.sync_copy(gb1, acc.at[didx.at[2 * k + 1]], add=True)
                return c

            lax.fori_loop(0, _STG // 2, step, 0)

        plsc.subcore_barrier()
        pltpu.sync_copy(acc.at[pl.ds(r0, _ROWS_PT)],
                        out_h.at[pl.ds(cid * _NP + r0, _ROWS_PT)])

    return body(table, src_ids, dst_ids, zeros_w)


def _tc_mm(x, w):
    m, k = x.shape
    n = w.shape[1]

    def kern(x_ref, w_ref, o_ref):
        o_ref[...] = jnp.dot(x_ref[...], w_ref[...],
                             preferred_element_type=jnp.float32)

    return pl.pallas_call(
        kern,
        grid=(m // _BLK,),
        in_specs=[
            pl.BlockSpec((_BLK, k), lambda i: (i, 0)),
            pl.BlockSpec((k, n), lambda i: (0, 0)),
        ],
        out_specs=pl.BlockSpec((_BLK, n), lambda i: (i, 0)),
        out_shape=jax.ShapeDtypeStruct((m, n), jnp.float32),
    )(x, w)


def _tc_combine(p, s, ea0, ea1, we_t, b, wn):
    m, h = p.shape
    n = wn.shape[1]

    def kern(p_ref, s_ref, e0_ref, e1_ref, we_ref, b_ref, wn_ref, o_ref):
        ecat = e0_ref[...] + e1_ref[...]
        eagg = ecat[:, :_DE]
        deg = ecat[:, _DE:_DE + 1]
        r = jnp.dot(eagg, we_ref[...], preferred_element_type=jnp.float32)
        hcur = jnp.maximum(deg * (p_ref[...] + b_ref[...])
                           + s_ref[...] + r, 0.0)
        o_ref[...] = jnp.dot(hcur, wn_ref[...],
                             preferred_element_type=jnp.float32)

    return pl.pallas_call(
        kern,
        grid=(m // _BLK,),
        in_specs=[
            pl.BlockSpec((_BLK, h), lambda i: (i, 0)),
            pl.BlockSpec((_BLK, h), lambda i: (i, 0)),
            pl.BlockSpec((_BLK, _WE), lambda i: (i, 0)),
            pl.BlockSpec((_BLK, _WE), lambda i: (i, 0)),
            pl.BlockSpec((_DE, h), lambda i: (0, 0)),
            pl.BlockSpec((1, h), lambda i: (0, 0)),
            pl.BlockSpec((h, n), lambda i: (0, 0)),
        ],
        out_specs=pl.BlockSpec((_BLK, n), lambda i: (i, 0)),
        out_shape=jax.ShapeDtypeStruct((m, n), jnp.float32),
    )(p, s, ea0, ea1, we_t, b, wn)


def _tc_final(p, s, ea0, ea1, we_t, b, batch, wl_t, bl):
    m, h = p.shape
    nblk = m // _BLK

    def kern(p_ref, s_ref, e0_ref, e1_ref, we_ref, b_ref, batch_ref, wl_ref,
             bl_ref, o_ref, sums, cnts):
        i = pl.program_id(0)

        @pl.when(i == 0)
        def _():
            sums[...] = jnp.zeros_like(sums)
            cnts[...] = jnp.zeros_like(cnts)

        ecat = e0_ref[...] + e1_ref[...]
        eagg = ecat[:, :_DE]
        deg = ecat[:, _DE:_DE + 1]
        r = jnp.dot(eagg, we_ref[...], preferred_element_type=jnp.float32)
        hcur = jnp.maximum(deg * (p_ref[...] + b_ref[...])
                           + s_ref[...] + r, 0.0)
        oh = (batch_ref[...][None, :]
              == lax.broadcasted_iota(jnp.int32, (_G, _BLK), 0)
              ).astype(jnp.float32)
        sums[...] += jnp.dot(oh, hcur, preferred_element_type=jnp.float32)
        cnts[...] += jnp.sum(oh, axis=1, keepdims=True)

        @pl.when(i == nblk - 1)
        def _():
            pooled = sums[...] / jnp.maximum(cnts[...], 1.0)
            logits = jnp.dot(pooled, wl_ref[...],
                             preferred_element_type=jnp.float32) + bl_ref[...]
            mx = jnp.max(logits, axis=1, keepdims=True)
            lse = jnp.log(jnp.sum(jnp.exp(logits - mx), axis=1, keepdims=True))
            o_ref[...] = (logits - mx) - lse

    return pl.pallas_call(
        kern,
        grid=(nblk,),
        in_specs=[
            pl.BlockSpec((_BLK, h), lambda i: (i, 0)),
            pl.BlockSpec((_BLK, h), lambda i: (i, 0)),
            pl.BlockSpec((_BLK, _WE), lambda i: (i, 0)),
            pl.BlockSpec((_BLK, _WE), lambda i: (i, 0)),
            pl.BlockSpec((_DE, h), lambda i: (0, 0)),
            pl.BlockSpec((1, h), lambda i: (0, 0)),
            pl.BlockSpec((_BLK,), lambda i: (i,)),
            pl.BlockSpec((h, _C), lambda i: (0, 0)),
            pl.BlockSpec((1, _C), lambda i: (0, 0)),
        ],
        out_specs=pl.BlockSpec((_G, _C), lambda i: (0, 0)),
        out_shape=jax.ShapeDtypeStruct((_G, _C), jnp.float32),
        scratch_shapes=[
            pltpu.VMEM((_G, h), jnp.float32),
            pltpu.VMEM((_G, 1), jnp.float32),
        ],
    )(p, s, ea0, ea1, we_t, b, batch, wl_t, bl)


def _mk_table(q):
    return jnp.concatenate([q[:, :_WS], q[:, _WS:]], axis=0)


def kernel(x, edge_index, edge_attr, batch, W1, b1, W2, b2, W3, b3, Wlin, blin):
    src = edge_index[0]
    dst = edge_index[1]

    xp = jnp.pad(x, ((0, _NP - _N), (0, 0)))
    batch_p = jnp.pad(batch, (0, _NP - _N), constant_values=_G)

    ept_real = _E // _NS
    src_t = jnp.pad(src.reshape(_NS, ept_real), ((0, 0), (0, _EPT - ept_real)))
    dst_t = jnp.pad(dst.reshape(_NS, ept_real), ((0, 0), (0, _EPT - ept_real)),
                    constant_values=_NP - 1)
    ea_aug = jnp.concatenate(
        [edge_attr, jnp.ones((_E, 1), jnp.float32),
         jnp.zeros((_E, _WE - _DE - 1), jnp.float32)], axis=1)
    ea_t = jnp.pad(ea_aug.reshape(_NS, ept_real, _WE),
                   ((0, 0), (0, _EPT - ept_real), (0, 0)))
    src_ids = jnp.concatenate([src_t, src_t + _NP], axis=0)
    src_ids = src_ids.reshape(_NC * _NS * _NCH, _CH)
    dst_ids = dst_t.reshape(_NS * _NCH, _CH)
    ea_ids = ea_t.reshape(_NS * _NCH, _CH, _WE)

    zeros_w = jnp.zeros((_NP, _WS), jnp.float32)
    zeros_e = jnp.zeros((_NP, _WE), jnp.float32)

    w1i, w1j, w1e = W1[:, :_DIN], W1[:, _DIN:2 * _DIN], W1[:, 2 * _DIN:]
    w2i, w2j, w2e = W2[:, :_H], W2[:, _H:2 * _H], W2[:, 2 * _H:]
    w3i, w3j, w3e = W3[:, :_H], W3[:, _H:2 * _H], W3[:, 2 * _H:]

    eacat = _sc_ea(ea_ids, dst_ids, zeros_e)
    ea0, ea1 = eacat[:_NP], eacat[_NP:]

    m1 = _tc_mm(xp, jnp.concatenate([w1i.T, w1j.T], axis=1))
    s1cat = _sc_spmm(_mk_table(m1[:, _H:]), src_ids, dst_ids, zeros_w)
    s1 = jnp.concatenate([s1cat[:_NP], s1cat[_NP:]], axis=1)

    m2 = _tc_combine(m1[:, :_H], s1, ea0, ea1, w1e.T, b1.reshape(1, _H),
                     jnp.concatenate([w2i.T, w2j.T], axis=1))
    s2cat = _sc_spmm(_mk_table(m2[:, _H:]), src_ids, dst_ids, zeros_w)
    s2 = jnp.concatenate([s2cat[:_NP], s2cat[_NP:]], axis=1)

    m3 = _tc_combine(m2[:, :_H], s2, ea0, ea1, w2e.T, b2.reshape(1, _H),
                     jnp.concatenate([w3i.T, w3j.T], axis=1))
    s3cat = _sc_spmm(_mk_table(m3[:, _H:]), src_ids, dst_ids, zeros_w)
    s3 = jnp.concatenate([s3cat[:_NP], s3cat[_NP:]], axis=1)

    return _tc_final(m3[:, :_H], s3, ea0, ea1, w3e.T, b3.reshape(1, _H),
                     batch_p, Wlin.T, blin.reshape(1, _C))
